# Initial kernel scaffold; baseline (speedup 1.0000x reference)
import jax
import jax.numpy as jnp
from jax.experimental import pallas as pl


def kernel(x, edge_index, W1, b1, W2, b2):
    raise NotImplementedError("write your pallas kernel here")


# speedup vs baseline: 8.2389x; 8.2389x over previous
import functools

import jax
import jax.numpy as jnp
from jax import lax
from jax.experimental import pallas as pl
from jax.experimental.pallas import tpu as pltpu
from jax.experimental.pallas import tpu_sc as plsc
from jax._src.pallas import mpmd as _mpmd

N_PAD = 10240
ROW_BLK = 256
K_BLK = 1024
CHUNK = 128
NW = 32


def _make_scatter(n_chunks):
    mesh = plsc.VectorSubcoreMesh(
        core_axis_name="c", subcore_axis_name="s", num_cores=2,
        num_subcores=16)

    def body(p_in_ref, idx_hbm, p_out_ref, idx_v, ones_v, sem):
        del p_in_ref
        w = lax.axis_index("s") * 2 + lax.axis_index("c")
        for i in range(CHUNK // 16):
            ones_v[pl.ds(i * 16, 16)] = jnp.full((16,), 1.0, jnp.float32)
        pltpu.sync_copy(idx_hbm.at[w], idx_v)
        copies = []
        for j in range(n_chunks):
            copies.append(
                pltpu.async_copy(ones_v, p_out_ref.at[idx_v.at[j]], sem))
        for c in copies:
            c.wait()

    return _mpmd._mpmd_map(
        [(mesh, body)],
        out_types=jax.ShapeDtypeStruct((N_PAD * N_PAD,), jnp.float32),
        input_output_aliases={0: 0},
        scratch_types=[
            pltpu.VMEM((n_chunks, CHUNK), jnp.int32),
            pltpu.VMEM((CHUNK,), jnp.float32),
            pltpu.SemaphoreType.DMA,
        ],
        name="edge_scatter",
    )


def _dis_body(p_ref, out_ref):
    k = pl.program_id(1)
    part = jnp.sum(p_ref[...], axis=1, keepdims=True)
    pb = jnp.broadcast_to(part, out_ref.shape)

    @pl.when(k == 0)
    def _():
        out_ref[...] = pb

    @pl.when(k > 0)
    def _():
        out_ref[...] += pb

    @pl.when(k == pl.num_programs(1) - 1)
    def _():
        out_ref[...] = lax.rsqrt(out_ref[...] + 1.0)


def _dis_call(p2d):
    grid = (N_PAD // ROW_BLK, N_PAD // K_BLK)
    return pl.pallas_call(
        _dis_body,
        grid=grid,
        in_specs=[pl.BlockSpec((ROW_BLK, K_BLK), lambda i, k: (i, k))],
        out_specs=pl.BlockSpec((ROW_BLK, 128), lambda i, k: (i, 0)),
        out_shape=jax.ShapeDtypeStruct((N_PAD, 128), jnp.float32),
        compiler_params=pltpu.CompilerParams(
            dimension_semantics=("parallel", "arbitrary")),
        name="rowsum_dis",
    )(p2d)


def _mm_t_body(x_ref, w_ref, dis_ref, out_ref):
    y = jnp.dot(x_ref[...], w_ref[...], preferred_element_type=jnp.float32)
    out_ref[...] = (y * dis_ref[:, :1]).astype(jnp.bfloat16)


def _mm_t_call(x, w, dis):
    f = x.shape[1]
    h = w.shape[1]
    grid = (N_PAD // ROW_BLK,)
    return pl.pallas_call(
        _mm_t_body,
        grid=grid,
        in_specs=[
            pl.BlockSpec((ROW_BLK, f), lambda i: (i, 0)),
            pl.BlockSpec((f, h), lambda i: (0, 0)),
            pl.BlockSpec((ROW_BLK, 128), lambda i: (i, 0)),
        ],
        out_specs=pl.BlockSpec((ROW_BLK, h), lambda i: (i, 0)),
        out_shape=jax.ShapeDtypeStruct((N_PAD, h), jnp.bfloat16),
        name="mm_t",
    )(x, w, dis)


def _gcn_body(relu, p_ref, t_ref, dis_ref, b_ref, out_ref):
    i = pl.program_id(0)
    k = pl.program_id(1)
    pb = p_ref[...].astype(jnp.bfloat16)
    tb = t_ref[pl.ds(k * K_BLK, K_BLK), :]
    z = jnp.dot(pb, tb, preferred_element_type=jnp.float32)

    @pl.when(k == 0)
    def _():
        out_ref[...] = z

    @pl.when(k > 0)
    def _():
        out_ref[...] += z

    @pl.when(k == pl.num_programs(1) - 1)
    def _():
        t_self = t_ref[pl.ds(i * ROW_BLK, ROW_BLK), :].astype(jnp.float32)
        r = (out_ref[...] + t_self) * dis_ref[:, :1] + b_ref[...]
        if relu:
            r = jnp.maximum(r, 0.0)
        out_ref[...] = r


def _gcn_call(p2d, t, dis, b, relu):
    h = t.shape[1]
    grid = (N_PAD // ROW_BLK, N_PAD // K_BLK)
    return pl.pallas_call(
        functools.partial(_gcn_body, relu),
        grid=grid,
        in_specs=[
            pl.BlockSpec((ROW_BLK, K_BLK), lambda i, k: (i, k)),
            pl.BlockSpec((N_PAD, h), lambda i, k: (0, 0)),
            pl.BlockSpec((ROW_BLK, 128), lambda i, k: (i, 0)),
            pl.BlockSpec((1, h), lambda i, k: (0, 0)),
        ],
        out_specs=pl.BlockSpec((ROW_BLK, h), lambda i, k: (i, 0)),
        out_shape=jax.ShapeDtypeStruct((N_PAD, h), jnp.float32),
        compiler_params=pltpu.CompilerParams(
            dimension_semantics=("parallel", "arbitrary")),
        name="gcn_layer",
    )(p2d, t, dis, b)


def kernel(x, edge_index, W1, b1, W2, b2):
    n = x.shape[0]
    e = edge_index.shape[1]
    per_w = -(-e // NW)
    n_chunks = -(-per_w // CHUNK)
    e_pad = NW * n_chunks * CHUNK

    ei = jnp.clip(edge_index, 0, n - 1)
    flat = ei[1].astype(jnp.int32) * N_PAD + ei[0].astype(jnp.int32)
    flat = jnp.concatenate(
        [flat, jnp.full((e_pad - e,), N_PAD * N_PAD - 1, jnp.int32)])
    flat = flat.reshape(NW, n_chunks, CHUNK)

    p0 = jnp.zeros((N_PAD * N_PAD,), jnp.float32)
    p = _make_scatter(n_chunks)(p0, flat)
    p2d = p.reshape(N_PAD, N_PAD)

    dis = _dis_call(p2d)

    xp = jnp.zeros((N_PAD, x.shape[1]), x.dtype).at[:n].set(x)
    t1 = _mm_t_call(xp, W1, dis)
    h = _gcn_call(p2d, t1, dis, b1.reshape(1, -1), relu=True)
    t2 = _mm_t_call(h, W2, dis)
    out = _gcn_call(p2d, t2, dis, b2.reshape(1, -1), relu=False)
    return out[:n]


# speedup vs baseline: 10.3111x; 1.2515x over previous
import functools

import jax
import jax.numpy as jnp
from jax import lax
from jax.experimental import pallas as pl
from jax.experimental.pallas import tpu as pltpu
from jax.experimental.pallas import tpu_sc as plsc
from jax._src.pallas import mpmd as _mpmd

N_PAD = 10240
ROW_BLK = 256
K_BLK = 1024
CHUNK = 128
NW = 32


def _make_scatter(per_w):
    mesh = plsc.VectorSubcoreMesh(
        core_axis_name="c", subcore_axis_name="s", num_cores=2,
        num_subcores=16)

    def body(p_in_ref, idx_hbm, ones_hbm, p_out_ref, idx_v, ones_v, sem):
        del p_in_ref
        w = lax.axis_index("s") * 2 + lax.axis_index("c")
        pltpu.sync_copy(idx_hbm.at[w], idx_v)
        pltpu.sync_copy(ones_hbm, ones_v)
        pltpu.async_copy(ones_v, p_out_ref.at[idx_v], sem).wait()

    return _mpmd._mpmd_map(
        [(mesh, body)],
        out_types=jax.ShapeDtypeStruct((N_PAD * N_PAD,), jnp.float32),
        input_output_aliases={0: 0},
        scratch_types=[
            pltpu.VMEM((per_w,), jnp.int32),
            pltpu.VMEM((per_w,), jnp.float32),
            pltpu.SemaphoreType.DMA,
        ],
        name="edge_scatter",
    )


def _dis_body(p_ref, out_ref, pbf_ref):
    k = pl.program_id(1)
    pblk = p_ref[...]
    pbf_ref[...] = pblk.astype(jnp.bfloat16)
    part = jnp.sum(pblk, axis=1, keepdims=True)
    pb = jnp.broadcast_to(part, out_ref.shape)

    @pl.when(k == 0)
    def _():
        out_ref[...] = pb

    @pl.when(k > 0)
    def _():
        out_ref[...] += pb

    @pl.when(k == pl.num_programs(1) - 1)
    def _():
        out_ref[...] = lax.rsqrt(out_ref[...] + 1.0)


def _dis_call(p2d):
    grid = (N_PAD // ROW_BLK, N_PAD // K_BLK)
    return pl.pallas_call(
        _dis_body,
        grid=grid,
        in_specs=[pl.BlockSpec((ROW_BLK, K_BLK), lambda i, k: (i, k))],
        out_specs=[
            pl.BlockSpec((ROW_BLK, 128), lambda i, k: (i, 0)),
            pl.BlockSpec((ROW_BLK, K_BLK), lambda i, k: (i, k)),
        ],
        out_shape=[
            jax.ShapeDtypeStruct((N_PAD, 128), jnp.float32),
            jax.ShapeDtypeStruct((N_PAD, N_PAD), jnp.bfloat16),
        ],
        compiler_params=pltpu.CompilerParams(
            dimension_semantics=("parallel", "arbitrary")),
        name="rowsum_dis",
    )(p2d)


def _mm_t_body(x_ref, w_ref, dis_ref, out_ref):
    y = jnp.dot(x_ref[...], w_ref[...], preferred_element_type=jnp.float32)
    out_ref[...] = (y * dis_ref[:, :1]).astype(jnp.bfloat16)


def _mm_t_call(x, w, dis):
    f = x.shape[1]
    h = w.shape[1]
    grid = (N_PAD // ROW_BLK,)
    return pl.pallas_call(
        _mm_t_body,
        grid=grid,
        in_specs=[
            pl.BlockSpec((ROW_BLK, f), lambda i: (i, 0)),
            pl.BlockSpec((f, h), lambda i: (0, 0)),
            pl.BlockSpec((ROW_BLK, 128), lambda i: (i, 0)),
        ],
        out_specs=pl.BlockSpec((ROW_BLK, h), lambda i: (i, 0)),
        out_shape=jax.ShapeDtypeStruct((N_PAD, h), jnp.bfloat16),
        name="mm_t",
    )(x, w, dis)


def _gcn_body(relu, p_ref, t_ref, dis_ref, b_ref, out_ref):
    i = pl.program_id(0)
    k = pl.program_id(1)
    pb = p_ref[...]
    tb = t_ref[pl.ds(k * K_BLK, K_BLK), :]
    z = jnp.dot(pb, tb, preferred_element_type=jnp.float32)

    @pl.when(k == 0)
    def _():
        out_ref[...] = z

    @pl.when(k > 0)
    def _():
        out_ref[...] += z

    @pl.when(k == pl.num_programs(1) - 1)
    def _():
        t_self = t_ref[pl.ds(i * ROW_BLK, ROW_BLK), :].astype(jnp.float32)
        r = (out_ref[...] + t_self) * dis_ref[:, :1] + b_ref[...]
        if relu:
            r = jnp.maximum(r, 0.0)
        out_ref[...] = r


def _gcn_call(p2d, t, dis, b, relu):
    h = t.shape[1]
    grid = (N_PAD // ROW_BLK, N_PAD // K_BLK)
    return pl.pallas_call(
        functools.partial(_gcn_body, relu),
        grid=grid,
        in_specs=[
            pl.BlockSpec((ROW_BLK, K_BLK), lambda i, k: (i, k)),
            pl.BlockSpec((N_PAD, h), lambda i, k: (0, 0)),
            pl.BlockSpec((ROW_BLK, 128), lambda i, k: (i, 0)),
            pl.BlockSpec((1, h), lambda i, k: (0, 0)),
        ],
        out_specs=pl.BlockSpec((ROW_BLK, h), lambda i, k: (i, 0)),
        out_shape=jax.ShapeDtypeStruct((N_PAD, h), jnp.float32),
        compiler_params=pltpu.CompilerParams(
            dimension_semantics=("parallel", "arbitrary")),
        name="gcn_layer",
    )(p2d, t, dis, b)


def kernel(x, edge_index, W1, b1, W2, b2):
    n = x.shape[0]
    e = edge_index.shape[1]
    per_w = -(-e // (NW * 32)) * 32
    e_pad = NW * per_w

    ei = jnp.clip(edge_index, 0, n - 1)
    flat = ei[1].astype(jnp.int32) * N_PAD + ei[0].astype(jnp.int32)
    flat = jnp.concatenate(
        [flat, jnp.full((e_pad - e,), N_PAD * N_PAD - 1, jnp.int32)])
    flat = flat.reshape(NW, per_w)
    ones = jnp.ones((per_w,), jnp.float32)

    p0 = jnp.zeros((N_PAD * N_PAD,), jnp.float32)
    p = _make_scatter(per_w)(p0, flat, ones)
    p2d = p.reshape(N_PAD, N_PAD)

    dis, pbf = _dis_call(p2d)

    xp = jnp.zeros((N_PAD, x.shape[1]), x.dtype).at[:n].set(x)
    t1 = _mm_t_call(xp, W1, dis)
    h = _gcn_call(pbf, t1, dis, b1.reshape(1, -1), relu=True)
    t2 = _mm_t_call(h, W2, dis)
    out = _gcn_call(pbf, t2, dis, b2.reshape(1, -1), relu=False)
    return out[:n]


# speedup vs baseline: 19.4162x; 1.8830x over previous
import functools

import jax
import jax.numpy as jnp
from jax import lax
from jax.experimental import pallas as pl
from jax.experimental.pallas import tpu as pltpu
from jax.experimental.pallas import tpu_sc as plsc
from jax._src.pallas import mpmd as _mpmd

N_PAD = 10240
ROW_BLK = 512
K_BLK = 2048
CHUNK = 128
NW = 32


def _make_scatter(per_w):
    mesh = plsc.VectorSubcoreMesh(
        core_axis_name="c", subcore_axis_name="s", num_cores=2,
        num_subcores=16)

    def body(p_in_ref, idx_hbm, ones_hbm, p_out_ref, idx_v, ones_v, sem):
        del p_in_ref
        w = lax.axis_index("s") * 2 + lax.axis_index("c")
        pltpu.sync_copy(idx_hbm.at[w], idx_v)
        pltpu.sync_copy(ones_hbm, ones_v)
        pltpu.async_copy(ones_v, p_out_ref.at[idx_v], sem).wait()

    return _mpmd._mpmd_map(
        [(mesh, body)],
        out_types=jax.ShapeDtypeStruct((N_PAD * N_PAD,), jnp.float32),
        input_output_aliases={0: 0},
        scratch_types=[
            pltpu.VMEM((per_w,), jnp.int32),
            pltpu.VMEM((per_w,), jnp.float32),
            pltpu.SemaphoreType.DMA,
        ],
        name="edge_scatter",
    )


DIS_ROWS = 256


def _dis_body(p_ref, out_ref, pbf_ref):
    pblk = p_ref[...].reshape(DIS_ROWS, N_PAD)
    pbf_ref[...] = pblk.astype(jnp.bfloat16)
    part = jnp.sum(pblk, axis=1, keepdims=True)
    pb = jnp.broadcast_to(part, out_ref.shape)
    out_ref[...] = lax.rsqrt(pb + 1.0)


def _dis_call(pflat):
    grid = (N_PAD // DIS_ROWS,)
    return pl.pallas_call(
        _dis_body,
        grid=grid,
        in_specs=[pl.BlockSpec((DIS_ROWS * N_PAD,), lambda i: (i,))],
        out_specs=[
            pl.BlockSpec((DIS_ROWS, 128), lambda i: (i, 0)),
            pl.BlockSpec((DIS_ROWS, N_PAD), lambda i: (i, 0)),
        ],
        out_shape=[
            jax.ShapeDtypeStruct((N_PAD, 128), jnp.float32),
            jax.ShapeDtypeStruct((N_PAD, N_PAD), jnp.bfloat16),
        ],
        name="rowsum_dis",
    )(pflat)


def _mm_t_body(x_ref, w_ref, dis_ref, out_ref):
    y = jnp.dot(x_ref[...], w_ref[...], preferred_element_type=jnp.float32)
    out_ref[...] = (y * dis_ref[:, :1]).astype(jnp.bfloat16)


def _mm_t_call(x, w, dis):
    f = x.shape[1]
    h = w.shape[1]
    grid = (N_PAD // ROW_BLK,)
    return pl.pallas_call(
        _mm_t_body,
        grid=grid,
        in_specs=[
            pl.BlockSpec((ROW_BLK, f), lambda i: (i, 0)),
            pl.BlockSpec((f, h), lambda i: (0, 0)),
            pl.BlockSpec((ROW_BLK, 128), lambda i: (i, 0)),
        ],
        out_specs=pl.BlockSpec((ROW_BLK, h), lambda i: (i, 0)),
        out_shape=jax.ShapeDtypeStruct((N_PAD, h), jnp.bfloat16),
        name="mm_t",
    )(x, w, dis)


def _gcn_body(relu, p_ref, t_ref, dis_ref, b_ref, out_ref):
    i = pl.program_id(0)
    k = pl.program_id(1)
    pb = p_ref[...]
    tb = t_ref[pl.ds(k * K_BLK, K_BLK), :]
    z = jnp.dot(pb, tb, preferred_element_type=jnp.float32)

    @pl.when(k == 0)
    def _():
        out_ref[...] = z

    @pl.when(k > 0)
    def _():
        out_ref[...] += z

    @pl.when(k == pl.num_programs(1) - 1)
    def _():
        t_self = t_ref[pl.ds(i * ROW_BLK, ROW_BLK), :].astype(jnp.float32)
        r = (out_ref[...] + t_self) * dis_ref[:, :1] + b_ref[...]
        if relu:
            r = jnp.maximum(r, 0.0)
        out_ref[...] = r


def _gcn_call(p2d, t, dis, b, relu):
    h = t.shape[1]
    grid = (N_PAD // ROW_BLK, N_PAD // K_BLK)
    return pl.pallas_call(
        functools.partial(_gcn_body, relu),
        grid=grid,
        in_specs=[
            pl.BlockSpec((ROW_BLK, K_BLK), lambda i, k: (i, k)),
            pl.BlockSpec((N_PAD, h), lambda i, k: (0, 0)),
            pl.BlockSpec((ROW_BLK, 128), lambda i, k: (i, 0)),
            pl.BlockSpec((1, h), lambda i, k: (0, 0)),
        ],
        out_specs=pl.BlockSpec((ROW_BLK, h), lambda i, k: (i, 0)),
        out_shape=jax.ShapeDtypeStruct((N_PAD, h), jnp.float32),
        compiler_params=pltpu.CompilerParams(
            dimension_semantics=("parallel", "arbitrary")),
        name="gcn_layer",
    )(p2d, t, dis, b)


def kernel(x, edge_index, W1, b1, W2, b2):
    n = x.shape[0]
    e = edge_index.shape[1]
    per_w = -(-e // (NW * 32)) * 32
    e_pad = NW * per_w

    ei = jnp.clip(edge_index, 0, n - 1)
    flat = ei[1].astype(jnp.int32) * N_PAD + ei[0].astype(jnp.int32)
    flat = jnp.concatenate(
        [flat, jnp.full((e_pad - e,), N_PAD * N_PAD - 1, jnp.int32)])
    flat = flat.reshape(NW, per_w)
    ones = jnp.ones((per_w,), jnp.float32)

    p0 = jnp.zeros((N_PAD * N_PAD,), jnp.float32)
    p = _make_scatter(per_w)(p0, flat, ones)

    dis, pbf = _dis_call(p)

    xp = jnp.zeros((N_PAD, x.shape[1]), x.dtype).at[:n].set(x)
    t1 = _mm_t_call(xp, W1, dis)
    h = _gcn_call(pbf, t1, dis, b1.reshape(1, -1), relu=True)
    t2 = _mm_t_call(h, W2, dis)
    out = _gcn_call(pbf, t2, dis, b2.reshape(1, -1), relu=False)
    return out[:n]


# speedup vs baseline: 19.4523x; 1.0019x over previous
import functools

import jax
import jax.numpy as jnp
from jax import lax
from jax.experimental import pallas as pl
from jax.experimental.pallas import tpu as pltpu
from jax.experimental.pallas import tpu_sc as plsc
from jax._src.pallas import mpmd as _mpmd

N_PAD = 10240
ROW_BLK = 512
K_BLK = 2048
CHUNK = 128
NW = 32


def _make_scatter(per_w):
    mesh = plsc.VectorSubcoreMesh(
        core_axis_name="c", subcore_axis_name="s", num_cores=2,
        num_subcores=16)

    def body(p_in_ref, idx_hbm, ones_hbm, p_out_ref, idx_v, ones_v, sem):
        del p_in_ref
        w = lax.axis_index("s") * 2 + lax.axis_index("c")
        pltpu.sync_copy(idx_hbm.at[w], idx_v)
        pltpu.sync_copy(ones_hbm, ones_v)
        pltpu.async_copy(ones_v, p_out_ref.at[idx_v], sem).wait()

    return _mpmd._mpmd_map(
        [(mesh, body)],
        out_types=jax.ShapeDtypeStruct((N_PAD * N_PAD,), jnp.float32),
        input_output_aliases={0: 0},
        scratch_types=[
            pltpu.VMEM((per_w,), jnp.int32),
            pltpu.VMEM((per_w,), jnp.float32),
            pltpu.SemaphoreType.DMA,
        ],
        name="edge_scatter",
    )


DIS_ROWS = 256


def _dis_body(p_ref, y_ref, out_ref, pbf_ref, t_ref):
    pblk = p_ref[...].reshape(DIS_ROWS, N_PAD)
    pbf_ref[...] = pblk.astype(jnp.bfloat16)
    part = jnp.sum(pblk, axis=1, keepdims=True)
    disv = lax.rsqrt(part + 1.0)
    out_ref[...] = jnp.broadcast_to(disv, out_ref.shape)
    t_ref[...] = (y_ref[...].astype(jnp.float32) * disv).astype(jnp.bfloat16)


def _dis_call(pflat, y):
    h = y.shape[1]
    grid = (N_PAD // DIS_ROWS,)
    return pl.pallas_call(
        _dis_body,
        grid=grid,
        in_specs=[
            pl.BlockSpec((DIS_ROWS * N_PAD,), lambda i: (i,)),
            pl.BlockSpec((DIS_ROWS, h), lambda i: (i, 0)),
        ],
        out_specs=[
            pl.BlockSpec((DIS_ROWS, 128), lambda i: (i, 0)),
            pl.BlockSpec((DIS_ROWS, N_PAD), lambda i: (i, 0)),
            pl.BlockSpec((DIS_ROWS, h), lambda i: (i, 0)),
        ],
        out_shape=[
            jax.ShapeDtypeStruct((N_PAD, 128), jnp.float32),
            jax.ShapeDtypeStruct((N_PAD, N_PAD), jnp.bfloat16),
            jax.ShapeDtypeStruct((N_PAD, h), jnp.bfloat16),
        ],
        name="rowsum_dis",
    )(pflat, y)


def _mm_t_body(x_ref, w_ref, dis_ref, out_ref):
    y = jnp.dot(x_ref[...], w_ref[...], preferred_element_type=jnp.float32)
    out_ref[...] = (y * dis_ref[:, :1]).astype(jnp.bfloat16)


def _mm_y_body(x_ref, w_ref, out_ref):
    y = jnp.dot(x_ref[...], w_ref[...], preferred_element_type=jnp.float32)
    out_ref[...] = y.astype(jnp.bfloat16)


def _mm_y_call(x, w):
    f = x.shape[1]
    h = w.shape[1]
    grid = (N_PAD // ROW_BLK,)
    return pl.pallas_call(
        _mm_y_body,
        grid=grid,
        in_specs=[
            pl.BlockSpec((ROW_BLK, f), lambda i: (i, 0)),
            pl.BlockSpec((f, h), lambda i: (0, 0)),
        ],
        out_specs=pl.BlockSpec((ROW_BLK, h), lambda i: (i, 0)),
        out_shape=jax.ShapeDtypeStruct((N_PAD, h), jnp.bfloat16),
        name="mm_y",
    )(x, w)


def _mm_t_call(x, w, dis):
    f = x.shape[1]
    h = w.shape[1]
    grid = (N_PAD // ROW_BLK,)
    return pl.pallas_call(
        _mm_t_body,
        grid=grid,
        in_specs=[
            pl.BlockSpec((ROW_BLK, f), lambda i: (i, 0)),
            pl.BlockSpec((f, h), lambda i: (0, 0)),
            pl.BlockSpec((ROW_BLK, 128), lambda i: (i, 0)),
        ],
        out_specs=pl.BlockSpec((ROW_BLK, h), lambda i: (i, 0)),
        out_shape=jax.ShapeDtypeStruct((N_PAD, h), jnp.bfloat16),
        name="mm_t",
    )(x, w, dis)


def _gcn_body(relu, p_ref, t_ref, dis_ref, b_ref, out_ref):
    i = pl.program_id(0)
    k = pl.program_id(1)
    pb = p_ref[...]
    tb = t_ref[pl.ds(k * K_BLK, K_BLK), :]
    z = jnp.dot(pb, tb, preferred_element_type=jnp.float32)

    @pl.when(k == 0)
    def _():
        out_ref[...] = z

    @pl.when(k > 0)
    def _():
        out_ref[...] += z

    @pl.when(k == pl.num_programs(1) - 1)
    def _():
        t_self = t_ref[pl.ds(i * ROW_BLK, ROW_BLK), :].astype(jnp.float32)
        r = (out_ref[...] + t_self) * dis_ref[:, :1] + b_ref[...]
        if relu:
            r = jnp.maximum(r, 0.0)
        out_ref[...] = r


def _gcn_call(p2d, t, dis, b, relu):
    h = t.shape[1]
    grid = (N_PAD // ROW_BLK, N_PAD // K_BLK)
    return pl.pallas_call(
        functools.partial(_gcn_body, relu),
        grid=grid,
        in_specs=[
            pl.BlockSpec((ROW_BLK, K_BLK), lambda i, k: (i, k)),
            pl.BlockSpec((N_PAD, h), lambda i, k: (0, 0)),
            pl.BlockSpec((ROW_BLK, 128), lambda i, k: (i, 0)),
            pl.BlockSpec((1, h), lambda i, k: (0, 0)),
        ],
        out_specs=pl.BlockSpec((ROW_BLK, h), lambda i, k: (i, 0)),
        out_shape=jax.ShapeDtypeStruct((N_PAD, h), jnp.float32),
        compiler_params=pltpu.CompilerParams(
            dimension_semantics=("parallel", "arbitrary")),
        name="gcn_layer",
    )(p2d, t, dis, b)


def kernel(x, edge_index, W1, b1, W2, b2):
    n = x.shape[0]
    e = edge_index.shape[1]
    per_w = -(-e // (NW * 32)) * 32
    e_pad = NW * per_w

    ei = jnp.clip(edge_index, 0, n - 1)
    flat = ei[1].astype(jnp.int32) * N_PAD + ei[0].astype(jnp.int32)
    flat = jnp.concatenate(
        [flat, jnp.full((e_pad - e,), N_PAD * N_PAD - 1, jnp.int32)])
    flat = flat.reshape(NW, per_w)
    ones = jnp.ones((per_w,), jnp.float32)

    p0 = jnp.zeros((N_PAD * N_PAD,), jnp.float32)
    p = _make_scatter(per_w)(p0, flat, ones)

    xp = jnp.zeros((N_PAD, x.shape[1]), x.dtype).at[:n].set(x)
    y1 = _mm_y_call(xp, W1)
    dis, pbf, t1 = _dis_call(p, y1)

    h = _gcn_call(pbf, t1, dis, b1.reshape(1, -1), relu=True)
    t2 = _mm_t_call(h, W2, dis)
    out = _gcn_call(pbf, t2, dis, b2.reshape(1, -1), relu=False)
    return out[:n]


# speedup vs baseline: 20.8800x; 1.0734x over previous
import functools

import jax
import jax.numpy as jnp
from jax import lax
from jax.experimental import pallas as pl
from jax.experimental.pallas import tpu as pltpu
from jax.experimental.pallas import tpu_sc as plsc
from jax._src.pallas import mpmd as _mpmd

N_PAD = 10240
ROW_BLK = 1024
K_BLK = 2048
CHUNK = 128
NW = 32


def _make_scatter(per_w):
    mesh = plsc.VectorSubcoreMesh(
        core_axis_name="c", subcore_axis_name="s", num_cores=2,
        num_subcores=16)

    def body(p_in_ref, idx_hbm, ones_hbm, p_out_ref, idx_v, ones_v, sem):
        del p_in_ref
        w = lax.axis_index("s") * 2 + lax.axis_index("c")
        pltpu.sync_copy(idx_hbm.at[w], idx_v)
        pltpu.sync_copy(ones_hbm, ones_v)
        pltpu.async_copy(ones_v, p_out_ref.at[idx_v], sem).wait()

    return _mpmd._mpmd_map(
        [(mesh, body)],
        out_types=jax.ShapeDtypeStruct((N_PAD * N_PAD,), jnp.float32),
        input_output_aliases={0: 0},
        scratch_types=[
            pltpu.VMEM((per_w,), jnp.int32),
            pltpu.VMEM((per_w,), jnp.float32),
            pltpu.SemaphoreType.DMA,
        ],
        name="edge_scatter",
    )


DIS_ROWS = 256


def _dis_body(p_ref, y_ref, out_ref, pbf_ref, t_ref):
    pblk = p_ref[...].reshape(DIS_ROWS, N_PAD)
    pbf_ref[...] = pblk.astype(jnp.bfloat16)
    part = jnp.sum(pblk, axis=1, keepdims=True)
    disv = lax.rsqrt(part + 1.0)
    out_ref[...] = jnp.broadcast_to(disv, out_ref.shape)
    t_ref[...] = (y_ref[...].astype(jnp.float32) * disv).astype(jnp.bfloat16)


def _dis_call(pflat, y):
    h = y.shape[1]
    grid = (N_PAD // DIS_ROWS,)
    return pl.pallas_call(
        _dis_body,
        grid=grid,
        in_specs=[
            pl.BlockSpec((DIS_ROWS * N_PAD,), lambda i: (i,)),
            pl.BlockSpec((DIS_ROWS, h), lambda i: (i, 0)),
        ],
        out_specs=[
            pl.BlockSpec((DIS_ROWS, 128), lambda i: (i, 0)),
            pl.BlockSpec((DIS_ROWS, N_PAD), lambda i: (i, 0)),
            pl.BlockSpec((DIS_ROWS, h), lambda i: (i, 0)),
        ],
        out_shape=[
            jax.ShapeDtypeStruct((N_PAD, 128), jnp.float32),
            jax.ShapeDtypeStruct((N_PAD, N_PAD), jnp.bfloat16),
            jax.ShapeDtypeStruct((N_PAD, h), jnp.bfloat16),
        ],
        name="rowsum_dis",
    )(pflat, y)


def _mm_t_body(x_ref, w_ref, dis_ref, out_ref):
    y = jnp.dot(x_ref[...], w_ref[...], preferred_element_type=jnp.float32)
    out_ref[...] = (y * dis_ref[:, :1]).astype(jnp.bfloat16)


def _mm_y_body(x_ref, w_ref, out_ref):
    y = jnp.dot(x_ref[...], w_ref[...], preferred_element_type=jnp.float32)
    out_ref[...] = y.astype(jnp.bfloat16)


def _mm_y_call(x, w):
    f = x.shape[1]
    h = w.shape[1]
    grid = (N_PAD // ROW_BLK,)
    return pl.pallas_call(
        _mm_y_body,
        grid=grid,
        in_specs=[
            pl.BlockSpec((ROW_BLK, f), lambda i: (i, 0)),
            pl.BlockSpec((f, h), lambda i: (0, 0)),
        ],
        out_specs=pl.BlockSpec((ROW_BLK, h), lambda i: (i, 0)),
        out_shape=jax.ShapeDtypeStruct((N_PAD, h), jnp.bfloat16),
        name="mm_y",
    )(x, w)


def _mm_t_call(x, w, dis):
    f = x.shape[1]
    h = w.shape[1]
    grid = (N_PAD // ROW_BLK,)
    return pl.pallas_call(
        _mm_t_body,
        grid=grid,
        in_specs=[
            pl.BlockSpec((ROW_BLK, f), lambda i: (i, 0)),
            pl.BlockSpec((f, h), lambda i: (0, 0)),
            pl.BlockSpec((ROW_BLK, 128), lambda i: (i, 0)),
        ],
        out_specs=pl.BlockSpec((ROW_BLK, h), lambda i: (i, 0)),
        out_shape=jax.ShapeDtypeStruct((N_PAD, h), jnp.bfloat16),
        name="mm_t",
    )(x, w, dis)


def _gcn_body(relu, p_ref, t_ref, dis_ref, b_ref, out_ref):
    i = pl.program_id(0)
    k = pl.program_id(1)
    pb = p_ref[...]
    tb = t_ref[pl.ds(k * K_BLK, K_BLK), :]
    z = jnp.dot(pb, tb, preferred_element_type=jnp.float32)

    @pl.when(k == 0)
    def _():
        out_ref[...] = z

    @pl.when(k > 0)
    def _():
        out_ref[...] += z

    @pl.when(k == pl.num_programs(1) - 1)
    def _():
        t_self = t_ref[pl.ds(i * ROW_BLK, ROW_BLK), :].astype(jnp.float32)
        r = (out_ref[...] + t_self) * dis_ref[:, :1] + b_ref[...]
        if relu:
            r = jnp.maximum(r, 0.0)
        out_ref[...] = r


def _gcn_call(p2d, t, dis, b, relu, out_rows=N_PAD):
    h = t.shape[1]
    grid = (N_PAD // ROW_BLK, N_PAD // K_BLK)
    return pl.pallas_call(
        functools.partial(_gcn_body, relu),
        grid=grid,
        in_specs=[
            pl.BlockSpec((ROW_BLK, K_BLK), lambda i, k: (i, k)),
            pl.BlockSpec((N_PAD, h), lambda i, k: (0, 0)),
            pl.BlockSpec((ROW_BLK, 128), lambda i, k: (i, 0)),
            pl.BlockSpec((1, h), lambda i, k: (0, 0)),
        ],
        out_specs=pl.BlockSpec((ROW_BLK, h), lambda i, k: (i, 0)),
        out_shape=jax.ShapeDtypeStruct((out_rows, h), jnp.float32),
        compiler_params=pltpu.CompilerParams(
            dimension_semantics=("parallel", "arbitrary")),
        name="gcn_layer",
    )(p2d, t, dis, b)


def kernel(x, edge_index, W1, b1, W2, b2):
    n = x.shape[0]
    e = edge_index.shape[1]
    per_w = -(-e // (NW * 32)) * 32
    e_pad = NW * per_w

    ei = jnp.clip(edge_index, 0, n - 1)
    flat = ei[1].astype(jnp.int32) * N_PAD + ei[0].astype(jnp.int32)
    flat = jnp.concatenate(
        [flat, jnp.full((e_pad - e,), N_PAD * N_PAD - 1, jnp.int32)])
    flat = flat.reshape(NW, per_w)
    ones = jnp.ones((per_w,), jnp.float32)

    p0 = jnp.zeros((N_PAD * N_PAD,), jnp.float32)
    p = _make_scatter(per_w)(p0, flat, ones)

    xp = jnp.zeros((N_PAD, x.shape[1]), x.dtype).at[:n].set(x)
    y1 = _mm_y_call(xp, W1)
    dis, pbf, t1 = _dis_call(p, y1)

    h = _gcn_call(pbf, t1, dis, b1.reshape(1, -1), relu=True)
    t2 = _mm_t_call(h, W2, dis)
    out = _gcn_call(pbf, t2, dis, b2.reshape(1, -1), relu=False, out_rows=n)
    return out
